# baseline (speedup 1.0000x reference)
import jax
import jax.numpy as jnp
from jax import lax
from jax.experimental import pallas as pl
from jax.experimental.pallas import tpu as pltpu
from jax.experimental.pallas import tpu_sc as plsc

NEG = -1.0e30
BIG = 1.0e30
B = 128
KC = 4
BLK = 512
BLK2 = 1000


def _tc_prep0(x_ref, w0_ref, as_ref, ad_ref, n_ref, h0_o, u_o, v_o, g_o, gsc):
    i = pl.program_id(0)
    blk = x_ref.shape[0]
    n_real = n_ref[0]
    h0 = jnp.dot(x_ref[...], w0_ref[...], preferred_element_type=jnp.float32)
    h0_o[...] = h0
    ps = h0 * as_ref[...]
    pd = h0 * ad_ref[...]
    H = u_o.shape[1] // 2
    C = h0.shape[1] // H
    asrc = jnp.concatenate(
        [jnp.sum(ps[:, h * C:(h + 1) * C], axis=1, keepdims=True) for h in range(H)], axis=1)
    adst = jnp.concatenate(
        [jnp.sum(pd[:, h * C:(h + 1) * C], axis=1, keepdims=True) for h in range(H)], axis=1)
    rowid = i * blk + lax.broadcasted_iota(jnp.int32, (blk, 1), 0)
    valid = rowid < n_real
    asrc = jnp.where(valid, asrc, NEG)
    adst = jnp.where(valid, adst, NEG)
    z8 = jnp.zeros((blk, H), jnp.float32)
    u_o[...] = jnp.concatenate([asrc, z8], axis=1)
    v_o[...] = jnp.concatenate([adst, z8], axis=1)

    @pl.when(i == 0)
    def _():
        gsc[...] = jnp.full((8, 128), NEG, jnp.float32)

    bm = jnp.max(asrc, axis=0, keepdims=True)
    gsc[0:1, 0:H] = jnp.maximum(gsc[0:1, 0:H], bm)
    g_o[...] = jnp.concatenate(
        [gsc[0:1, 0:H], jnp.full((1, 16 - H), BIG, jnp.float32)], axis=1)


def _tc_mid(a0_ref, a1_ref, wa0_ref, wa1_ref, b0_ref, w1_ref, as1_ref, ad1_ref,
            h1e_o, adt_o, g1_o, gsc):
    i = pl.program_id(0)
    blk = a0_ref.shape[0]
    s = a0_ref[...] + a1_ref[...]
    den = wa0_ref[...] + wa1_ref[...]
    HC = b0_ref.shape[1]
    H = wa0_ref.shape[1] // 2
    C = HC // H
    outs = []
    for h in range(H):
        outs.append(s[:, h * C:(h + 1) * C] / (den[:, h:h + 1] + 1e-16))
    z = jnp.concatenate(outs, axis=1) + b0_ref[...]
    hp = jnp.where(z > 0, z, jnp.exp(jnp.minimum(z, 0.0)) - 1.0)
    h1 = jnp.dot(hp, w1_ref[...], preferred_element_type=jnp.float32)
    as1 = jnp.sum(h1 * as1_ref[...], axis=1, keepdims=True)
    ad1 = jnp.sum(h1 * ad1_ref[...], axis=1, keepdims=True)
    h1e_o[...] = jnp.concatenate(
        [h1, jnp.ones((blk, 1), jnp.float32), jnp.zeros((blk, 15), jnp.float32)], axis=1)
    adt_o[...] = jnp.concatenate(
        [as1, ad1, jnp.zeros((blk, 14), jnp.float32)], axis=1)

    @pl.when(i == 0)
    def _():
        gsc[...] = jnp.full((8, 128), NEG, jnp.float32)

    gsc[0:1, 0:1] = jnp.maximum(gsc[0:1, 0:1], jnp.max(as1, axis=0, keepdims=True))
    g1_o[...] = jnp.broadcast_to(gsc[0:1, 0:1], (1, 16))


def _tc_final(a0_ref, a1_ref, b1_ref, out_o):
    s = a0_ref[...] + a1_ref[...]
    OC = b1_ref.shape[1]
    out_o[...] = s[:, 0:OC] / (s[:, OC:OC + 1] + 1e-16) + b1_ref[...]


def _sc_edge0(n, ka, kb):
    rows_pt = n // 16

    ga, gb = ka // KC, kb // KC

    def body(src_hbm, dst_hbm, u_hbm, v_hbm, h0_hbm, g_hbm, z128_hbm, z16_hbm,
             outh_hbm, outw_hbm,
             acch, accw, gv, six0, dix0, six1, dix1, ubuf, vbuf, wbuf,
             hbuf0, hbuf1, suv, sh0, sh1, sgi):
        cid = lax.axis_index("c")
        sid = lax.axis_index("s")
        chunks = jnp.where(cid == 0, ka, kb)
        groups = jnp.where(cid == 0, ga, gb)
        cbase = jnp.where(cid == 0, sid * ka, 16 * ka + sid * kb)
        r0 = sid * rows_pt

        pltpu.sync_copy(g_hbm, gv)
        pltpu.sync_copy(src_hbm.at[pl.ds(cbase, KC)], six0)
        pltpu.sync_copy(dst_hbm.at[pl.ds(cbase, KC)], dix0)
        pltpu.async_copy(src_hbm.at[pl.ds(cbase + KC, KC)], six1, sgi)
        pltpu.async_copy(dst_hbm.at[pl.ds(cbase + KC, KC)], dix1, sgi)
        pltpu.async_copy(u_hbm.at[six0.at[0]], ubuf, suv)
        pltpu.async_copy(v_hbm.at[dix0.at[0]], vbuf, suv)
        pltpu.async_copy(h0_hbm.at[six0.at[0]], hbuf0, sh0)
        pltpu.async_copy(h0_hbm.at[six0.at[1]], hbuf1, sh1)

        pltpu.sync_copy(z128_hbm.at[pl.ds(r0, rows_pt)], acch.at[pl.ds(r0, rows_pt)])
        pltpu.sync_copy(z16_hbm.at[pl.ds(r0, rows_pt)], accw.at[pl.ds(r0, rows_pt)])
        plsc.subcore_barrier()

        g = gv[...]

        def phase(c, sb, db, hb, sh, sb1, db1, sb2):
            pltpu.make_async_copy(u_hbm.at[sb], ubuf, suv).wait()
            pltpu.make_async_copy(v_hbm.at[db], vbuf, suv).wait()

            @plsc.parallel_loop(0, B, unroll=4)
            def wcalc(e):
                uz = ubuf[e]
                vz = vbuf[e]
                zz = uz + vz
                lz = jnp.where(zz > 0, zz, 0.2 * zz)
                m = jnp.maximum(g + vz, 0.0)
                wbuf[e] = jnp.exp(lz - m)

            @pl.when(c + 1 < chunks)
            def _():
                pltpu.async_copy(u_hbm.at[sb1], ubuf, suv)
                pltpu.async_copy(v_hbm.at[db1], vbuf, suv)

            pltpu.make_async_copy(h0_hbm.at[sb], hb, sh).wait()

            @plsc.parallel_loop(0, B, unroll=2)
            def mcalc(e):
                wvec = wbuf[e]
                for j in range(8):
                    hb[e, pl.ds(j * 16, 16)] = wvec[j] * hb[e, pl.ds(j * 16, 16)]

            pltpu.sync_copy(hb, acch.at[db], add=True)
            pltpu.sync_copy(wbuf, accw.at[db], add=True)

            @pl.when(c + 2 < chunks)
            def _():
                pltpu.async_copy(h0_hbm.at[sb2], hb, sh)

        def group(gi, sixa, dixa, sixb, dixb):
            for j in range(KC):
                c = gi * KC + j
                if j == KC - 2:
                    @pl.when(gi + 1 < groups)
                    def _():
                        pltpu.make_async_copy(
                            src_hbm.at[pl.ds(cbase + (gi + 1) * KC, KC)], sixb, sgi).wait()
                        pltpu.make_async_copy(
                            dst_hbm.at[pl.ds(cbase + (gi + 1) * KC, KC)], dixb, sgi).wait()
                sb, db = sixa.at[j], dixa.at[j]
                sb1 = sixa.at[j + 1] if j + 1 < KC else sixb.at[0]
                db1 = dixa.at[j + 1] if j + 1 < KC else dixb.at[0]
                sb2 = sixa.at[j + 2] if j + 2 < KC else sixb.at[j + 2 - KC]
                hb, sh = (hbuf0, sh0) if j % 2 == 0 else (hbuf1, sh1)
                phase(c, sb, db, hb, sh, sb1, db1, sb2)

            @pl.when(gi + 2 < groups)
            def _():
                pltpu.async_copy(
                    src_hbm.at[pl.ds(cbase + (gi + 2) * KC, KC)], sixa, sgi)
                pltpu.async_copy(
                    dst_hbm.at[pl.ds(cbase + (gi + 2) * KC, KC)], dixa, sgi)

        def gpair(p, _):
            group(2 * p, six0, dix0, six1, dix1)
            group(2 * p + 1, six1, dix1, six0, dix0)
            return 0
        lax.fori_loop(0, jnp.where(cid == 0, ga // 2, gb // 2), gpair, 0)

        plsc.subcore_barrier()
        pltpu.sync_copy(acch.at[pl.ds(r0, rows_pt)],
                        outh_hbm.at[pl.ds(cid * n + r0, rows_pt)])
        pltpu.sync_copy(accw.at[pl.ds(r0, rows_pt)],
                        outw_hbm.at[pl.ds(cid * n + r0, rows_pt)])

    return pl.kernel(
        body,
        out_type=(jax.ShapeDtypeStruct((2 * n, 128), jnp.float32),
                  jax.ShapeDtypeStruct((2 * n, 16), jnp.float32)),
        compiler_params=pltpu.CompilerParams(
            use_tc_tiling_on_sc=False, needs_layout_passes=False),
        mesh=plsc.VectorSubcoreMesh(core_axis_name="c", subcore_axis_name="s"),
        scratch_types=[
            pltpu.VMEM_SHARED((n, 128), jnp.float32),
            pltpu.VMEM_SHARED((n, 16), jnp.float32),
            pltpu.VMEM((16,), jnp.float32),
            pltpu.VMEM((KC, B), jnp.int32),
            pltpu.VMEM((KC, B), jnp.int32),
            pltpu.VMEM((KC, B), jnp.int32),
            pltpu.VMEM((KC, B), jnp.int32),
            pltpu.VMEM((B, 16), jnp.float32),
            pltpu.VMEM((B, 16), jnp.float32),
            pltpu.VMEM((B, 16), jnp.float32),
            pltpu.VMEM((B, 128), jnp.float32),
            pltpu.VMEM((B, 128), jnp.float32),
            pltpu.SemaphoreType.DMA,
            pltpu.SemaphoreType.DMA,
            pltpu.SemaphoreType.DMA,
            pltpu.SemaphoreType.DMA,
        ],
    )


def _sc_edge1(n, np_, ka, kb):
    rows_pt = n // 16

    ga, gb = ka // KC, kb // KC

    def body(src_hbm, dst_hbm, a1_hbm, d1_hbm, h1e_hbm, g_hbm, z80_hbm, out_hbm,
             accs, gv, a1v, d1v, six0, dix0, six1, dix1, wbuf,
             mbuf0, mbuf1, sh0, sh1, sgi):
        cid = lax.axis_index("c")
        sid = lax.axis_index("s")
        chunks = jnp.where(cid == 0, ka, kb)
        groups = jnp.where(cid == 0, ga, gb)
        cbase = jnp.where(cid == 0, sid * ka, 16 * ka + sid * kb)
        r0 = sid * rows_pt

        pltpu.sync_copy(g_hbm, gv)
        pltpu.sync_copy(src_hbm.at[pl.ds(cbase, KC)], six0)
        pltpu.sync_copy(dst_hbm.at[pl.ds(cbase, KC)], dix0)
        pltpu.async_copy(src_hbm.at[pl.ds(cbase + KC, KC)], six1, sgi)
        pltpu.async_copy(dst_hbm.at[pl.ds(cbase + KC, KC)], dix1, sgi)
        pltpu.async_copy(h1e_hbm.at[six0.at[0]], mbuf0, sh0)
        pltpu.async_copy(h1e_hbm.at[six0.at[1]], mbuf1, sh1)
        pltpu.sync_copy(a1_hbm, a1v)
        pltpu.sync_copy(d1_hbm, d1v)

        pltpu.sync_copy(z80_hbm.at[pl.ds(r0, rows_pt)], accs.at[pl.ds(r0, rows_pt)])
        plsc.subcore_barrier()

        g = gv[...]

        def phase(c, sb, db, mb, sh, sb2):
            @plsc.parallel_loop(0, B // 16, unroll=2)
            def wcalc(q):
                srcv = sb[pl.ds(q * 16, 16)]
                dstv = db[pl.ds(q * 16, 16)]
                a = plsc.load_gather(a1v, [srcv])
                d = plsc.load_gather(d1v, [dstv])
                zz = a + d
                lz = jnp.where(zz > 0, zz, 0.2 * zz)
                m = jnp.maximum(g + d, 0.0)
                wbuf[pl.ds(q * 16, 16)] = jnp.exp(lz - m)

            pltpu.make_async_copy(h1e_hbm.at[sb], mb, sh).wait()

            @plsc.parallel_loop(0, B // 16)
            def mcalc(q):
                wvec = wbuf[pl.ds(q * 16, 16)]
                for i in range(16):
                    e = q * 16 + i
                    ws = wvec[i]
                    for j in range(5):
                        mb[e, pl.ds(j * 16, 16)] = ws * mb[e, pl.ds(j * 16, 16)]

            pltpu.sync_copy(mb, accs.at[db], add=True)

            @pl.when(c + 2 < chunks)
            def _():
                pltpu.async_copy(h1e_hbm.at[sb2], mb, sh)

        def group(gi, sixa, dixa, sixb, dixb):
            for j in range(KC):
                c = gi * KC + j
                if j == KC - 2:
                    @pl.when(gi + 1 < groups)
                    def _():
                        pltpu.make_async_copy(
                            src_hbm.at[pl.ds(cbase + (gi + 1) * KC, KC)], sixb, sgi).wait()
                        pltpu.make_async_copy(
                            dst_hbm.at[pl.ds(cbase + (gi + 1) * KC, KC)], dixb, sgi).wait()
                sb, db = sixa.at[j], dixa.at[j]
                sb2 = sixa.at[j + 2] if j + 2 < KC else sixb.at[j + 2 - KC]
                mb, sh = (mbuf0, sh0) if j % 2 == 0 else (mbuf1, sh1)
                phase(c, sb, db, mb, sh, sb2)

            @pl.when(gi + 2 < groups)
            def _():
                pltpu.async_copy(
                    src_hbm.at[pl.ds(cbase + (gi + 2) * KC, KC)], sixa, sgi)
                pltpu.async_copy(
                    dst_hbm.at[pl.ds(cbase + (gi + 2) * KC, KC)], dixa, sgi)

        def gpair(p, _):
            group(2 * p, six0, dix0, six1, dix1)
            group(2 * p + 1, six1, dix1, six0, dix0)
            return 0
        lax.fori_loop(0, jnp.where(cid == 0, ga // 2, gb // 2), gpair, 0)

        plsc.subcore_barrier()
        pltpu.sync_copy(accs.at[pl.ds(r0, rows_pt)],
                        out_hbm.at[pl.ds(cid * n + r0, rows_pt)])

    return pl.kernel(
        body,
        out_type=jax.ShapeDtypeStruct((2 * n, 80), jnp.float32),
        compiler_params=pltpu.CompilerParams(
            use_tc_tiling_on_sc=False, needs_layout_passes=False),
        mesh=plsc.VectorSubcoreMesh(core_axis_name="c", subcore_axis_name="s"),
        scratch_types=[
            pltpu.VMEM_SHARED((n, 80), jnp.float32),
            pltpu.VMEM((16,), jnp.float32),
            pltpu.VMEM((np_,), jnp.float32),
            pltpu.VMEM((np_,), jnp.float32),
            pltpu.VMEM((KC, B), jnp.int32),
            pltpu.VMEM((KC, B), jnp.int32),
            pltpu.VMEM((KC, B), jnp.int32),
            pltpu.VMEM((KC, B), jnp.int32),
            pltpu.VMEM((B,), jnp.float32),
            pltpu.VMEM((B, 80), jnp.float32),
            pltpu.VMEM((B, 80), jnp.float32),
            pltpu.SemaphoreType.DMA,
            pltpu.SemaphoreType.DMA,
            pltpu.SemaphoreType.DMA,
        ],
    )


def kernel(x, edge_index, W0, a_src0, a_dst0, b0, W1, a_src1, a_dst1, b1):
    N, IN = x.shape
    HC = W0.shape[1]
    H = a_src0.shape[1]
    OC = W1.shape[1]
    f32 = jnp.float32

    np_ = ((N + 1023) // 1024) * 1024
    nblk = np_ // BLK
    nblk2 = N // BLK2

    ei = edge_index.astype(jnp.int32)
    loop = jnp.arange(N, dtype=jnp.int32)
    src = jnp.concatenate([ei[0], loop])
    dst = jnp.concatenate([ei[1], loop])
    etot = src.shape[0]
    step = 16 * B * 2 * KC
    ep = ((etot + step - 1) // step) * step
    tot = ep // (16 * B)

    def _split(frac):
        m = 2 * KC
        ka = min(tot - m, max(m, int(round(tot * frac / m)) * m))
        return ka, tot - ka

    ka0, kb0 = _split(0.646)
    ka1, kb1 = _split(0.60)
    pad = ep - etot
    src = jnp.concatenate([src, jnp.full((pad,), np_ - 1, jnp.int32)]).reshape(-1, B)
    dst = jnp.concatenate([dst, jnp.zeros((pad,), jnp.int32)]).reshape(-1, B)

    xp = jnp.pad(x, ((0, np_ - N), (0, 0)))
    n_arr = jnp.array([N], jnp.int32)
    z128 = jnp.zeros((N, 128), f32)
    z16 = jnp.zeros((N, 16), f32)
    z80 = jnp.zeros((N, 80), f32)

    h0, U, V, g16 = pl.pallas_call(
        _tc_prep0,
        grid=(nblk,),
        in_specs=[
            pl.BlockSpec((BLK, IN), lambda i: (i, 0)),
            pl.BlockSpec((IN, HC), lambda i: (0, 0)),
            pl.BlockSpec((1, HC), lambda i: (0, 0)),
            pl.BlockSpec((1, HC), lambda i: (0, 0)),
            pl.BlockSpec(memory_space=pltpu.SMEM),
        ],
        out_specs=[
            pl.BlockSpec((BLK, HC), lambda i: (i, 0)),
            pl.BlockSpec((BLK, 16), lambda i: (i, 0)),
            pl.BlockSpec((BLK, 16), lambda i: (i, 0)),
            pl.BlockSpec((1, 16), lambda i: (0, 0)),
        ],
        out_shape=[
            jax.ShapeDtypeStruct((np_, HC), f32),
            jax.ShapeDtypeStruct((np_, 16), f32),
            jax.ShapeDtypeStruct((np_, 16), f32),
            jax.ShapeDtypeStruct((1, 16), f32),
        ],
        scratch_shapes=[pltpu.VMEM((8, 128), f32)],
    )(xp, W0, a_src0.reshape(1, HC), a_dst0.reshape(1, HC), n_arr)

    acch, accw = _sc_edge0(N, ka0, kb0)(
        src, dst, U, V, h0, g16.reshape(16), z128, z16)

    h1e, adt, g1 = pl.pallas_call(
        _tc_mid,
        grid=(nblk2,),
        in_specs=[
            pl.BlockSpec((BLK2, HC), lambda i: (i, 0)),
            pl.BlockSpec((BLK2, HC), lambda i, nb=nblk2: (i + nb, 0)),
            pl.BlockSpec((BLK2, 16), lambda i: (i, 0)),
            pl.BlockSpec((BLK2, 16), lambda i, nb=nblk2: (i + nb, 0)),
            pl.BlockSpec((1, HC), lambda i: (0, 0)),
            pl.BlockSpec((HC, OC), lambda i: (0, 0)),
            pl.BlockSpec((1, OC), lambda i: (0, 0)),
            pl.BlockSpec((1, OC), lambda i: (0, 0)),
        ],
        out_specs=[
            pl.BlockSpec((BLK2, 80), lambda i: (i, 0)),
            pl.BlockSpec((BLK2, 16), lambda i: (i, 0)),
            pl.BlockSpec((1, 16), lambda i: (0, 0)),
        ],
        out_shape=[
            jax.ShapeDtypeStruct((N, 80), f32),
            jax.ShapeDtypeStruct((N, 16), f32),
            jax.ShapeDtypeStruct((1, 16), f32),
        ],
        scratch_shapes=[pltpu.VMEM((8, 128), f32)],
    )(acch, acch, accw, accw, b0.reshape(1, HC), W1, a_src1.reshape(1, OC),
      a_dst1.reshape(1, OC))

    h1e_p = jnp.pad(h1e, ((0, np_ - N), (0, 0)))
    a1t = jnp.pad(adt[:, 0].reshape(N), (0, np_ - N), constant_values=NEG)
    d1t = jnp.pad(adt[:, 1].reshape(N), (0, np_ - N))
    acc1 = _sc_edge1(N, np_, ka1, kb1)(
        src, dst, a1t, d1t, h1e_p, g1.reshape(16), z80)

    out = pl.pallas_call(
        _tc_final,
        grid=(nblk2,),
        in_specs=[
            pl.BlockSpec((BLK2, 80), lambda i: (i, 0)),
            pl.BlockSpec((BLK2, 80), lambda i, nb=nblk2: (i + nb, 0)),
            pl.BlockSpec((1, OC), lambda i: (0, 0)),
        ],
        out_specs=pl.BlockSpec((BLK2, OC), lambda i: (i, 0)),
        out_shape=jax.ShapeDtypeStruct((N, OC), f32),
    )(acc1, acc1, b1.reshape(1, OC))

    return out


# speedup vs baseline: 1.0279x; 1.0279x over previous
import jax
import jax.numpy as jnp
from jax import lax
from jax.experimental import pallas as pl
from jax.experimental.pallas import tpu as pltpu
from jax.experimental.pallas import tpu_sc as plsc

NEG = -1.0e30
BIG = 1.0e30
B = 128
KC = 4
BLK = 512
BLK2 = 1000


def _tc_prep0(x_ref, w0_ref, as_ref, ad_ref, n_ref, h0_o, u_o, v_o, g_o, gsc):
    i = pl.program_id(0)
    blk = x_ref.shape[0]
    n_real = n_ref[0]
    h0 = jnp.dot(x_ref[...], w0_ref[...], preferred_element_type=jnp.float32)
    h0_o[...] = h0
    ps = h0 * as_ref[...]
    pd = h0 * ad_ref[...]
    H = u_o.shape[1] // 2
    C = h0.shape[1] // H
    asrc = jnp.concatenate(
        [jnp.sum(ps[:, h * C:(h + 1) * C], axis=1, keepdims=True) for h in range(H)], axis=1)
    adst = jnp.concatenate(
        [jnp.sum(pd[:, h * C:(h + 1) * C], axis=1, keepdims=True) for h in range(H)], axis=1)
    rowid = i * blk + lax.broadcasted_iota(jnp.int32, (blk, 1), 0)
    valid = rowid < n_real
    asrc = jnp.where(valid, asrc, NEG)
    adst = jnp.where(valid, adst, NEG)
    z8 = jnp.zeros((blk, H), jnp.float32)
    u_o[...] = jnp.concatenate([asrc, z8], axis=1)
    v_o[...] = jnp.concatenate([adst, z8], axis=1)

    @pl.when(i == 0)
    def _():
        gsc[...] = jnp.full((8, 128), NEG, jnp.float32)

    bm = jnp.max(asrc, axis=0, keepdims=True)
    gsc[0:1, 0:H] = jnp.maximum(gsc[0:1, 0:H], bm)
    g_o[...] = jnp.concatenate(
        [gsc[0:1, 0:H], jnp.full((1, 16 - H), BIG, jnp.float32)], axis=1)


def _tc_mid(a0_ref, a1_ref, wa0_ref, wa1_ref, b0_ref, w1_ref, as1_ref, ad1_ref,
            h1e_o, adt_o, g1_o, gsc):
    i = pl.program_id(0)
    blk = a0_ref.shape[0]
    s = a0_ref[...] + a1_ref[...]
    den = wa0_ref[...] + wa1_ref[...]
    HC = b0_ref.shape[1]
    H = wa0_ref.shape[1] // 2
    C = HC // H
    outs = []
    for h in range(H):
        outs.append(s[:, h * C:(h + 1) * C] / (den[:, h:h + 1] + 1e-16))
    z = jnp.concatenate(outs, axis=1) + b0_ref[...]
    hp = jnp.where(z > 0, z, jnp.exp(jnp.minimum(z, 0.0)) - 1.0)
    h1 = jnp.dot(hp, w1_ref[...], preferred_element_type=jnp.float32)
    as1 = jnp.sum(h1 * as1_ref[...], axis=1, keepdims=True)
    ad1 = jnp.sum(h1 * ad1_ref[...], axis=1, keepdims=True)
    h1e_o[...] = jnp.concatenate(
        [h1, jnp.ones((blk, 1), jnp.float32), jnp.zeros((blk, 15), jnp.float32)], axis=1)
    adt_o[...] = jnp.concatenate(
        [as1, ad1, jnp.zeros((blk, 14), jnp.float32)], axis=1)

    @pl.when(i == 0)
    def _():
        gsc[...] = jnp.full((8, 128), NEG, jnp.float32)

    gsc[0:1, 0:1] = jnp.maximum(gsc[0:1, 0:1], jnp.max(as1, axis=0, keepdims=True))
    g1_o[...] = jnp.broadcast_to(gsc[0:1, 0:1], (1, 16))


def _tc_final(a0_ref, a1_ref, b1_ref, out_o):
    s = a0_ref[...] + a1_ref[...]
    OC = b1_ref.shape[1]
    out_o[...] = s[:, 0:OC] / (s[:, OC:OC + 1] + 1e-16) + b1_ref[...]


def _sc_edge0(n, ka, kb):
    rows_pt = n // 16

    def body(src_hbm, dst_hbm, u_hbm, v_hbm, h0_hbm, g_hbm, z128_hbm, z16_hbm,
             outh_hbm, outw_hbm,
             acch, accw, gv, six0, dix0, six1, dix1, ubuf, vbuf, wbuf,
             hbuf0, hbuf1, suv, sh0, sh1):
        cid = lax.axis_index("c")
        sid = lax.axis_index("s")
        chunks = jnp.where(cid == 0, ka, kb)
        cbase = jnp.where(cid == 0, sid * ka, 16 * ka + sid * kb)
        r0 = sid * rows_pt

        pltpu.sync_copy(g_hbm, gv)
        pltpu.sync_copy(src_hbm.at[cbase], six0)
        pltpu.sync_copy(dst_hbm.at[cbase], dix0)
        pltpu.sync_copy(src_hbm.at[cbase + 1], six1)
        pltpu.sync_copy(dst_hbm.at[cbase + 1], dix1)
        pltpu.async_copy(u_hbm.at[six0], ubuf, suv)
        pltpu.async_copy(v_hbm.at[dix0], vbuf, suv)
        pltpu.async_copy(h0_hbm.at[six0], hbuf0, sh0)
        pltpu.async_copy(h0_hbm.at[six1], hbuf1, sh1)

        pltpu.sync_copy(z128_hbm.at[pl.ds(r0, rows_pt)], acch.at[pl.ds(r0, rows_pt)])
        pltpu.sync_copy(z16_hbm.at[pl.ds(r0, rows_pt)], accw.at[pl.ds(r0, rows_pt)])
        plsc.subcore_barrier()

        g = gv[...]

        def phase(c, sb, db, hb, sh, sb_o, db_o):
            pltpu.make_async_copy(u_hbm.at[sb], ubuf, suv).wait()
            pltpu.make_async_copy(v_hbm.at[db], vbuf, suv).wait()

            @plsc.parallel_loop(0, B, unroll=4)
            def wcalc(e):
                uz = ubuf[e]
                vz = vbuf[e]
                zz = uz + vz
                lz = jnp.where(zz > 0, zz, 0.2 * zz)
                m = jnp.maximum(g + vz, 0.0)
                wbuf[e] = jnp.exp(lz - m)

            @pl.when(c + 1 < chunks)
            def _():
                pltpu.async_copy(u_hbm.at[sb_o], ubuf, suv)
                pltpu.async_copy(v_hbm.at[db_o], vbuf, suv)

            pltpu.make_async_copy(h0_hbm.at[sb], hb, sh).wait()

            @plsc.parallel_loop(0, B, unroll=2)
            def mcalc(e):
                wvec = wbuf[e]
                for j in range(8):
                    hb[e, pl.ds(j * 16, 16)] = wvec[j] * hb[e, pl.ds(j * 16, 16)]

            pltpu.sync_copy(hb, acch.at[db], add=True)
            pltpu.sync_copy(wbuf, accw.at[db], add=True)

            @pl.when(c + 2 < chunks)
            def _():
                pltpu.sync_copy(src_hbm.at[cbase + c + 2], sb)
                pltpu.sync_copy(dst_hbm.at[cbase + c + 2], db)
                pltpu.async_copy(h0_hbm.at[sb], hb, sh)

        def pair(p, _):
            c0 = 2 * p
            phase(c0, six0, dix0, hbuf0, sh0, six1, dix1)
            phase(c0 + 1, six1, dix1, hbuf1, sh1, six0, dix0)
            return 0
        lax.fori_loop(0, jnp.where(cid == 0, ka // 2, kb // 2), pair, 0)

        plsc.subcore_barrier()
        pltpu.sync_copy(acch.at[pl.ds(r0, rows_pt)],
                        outh_hbm.at[pl.ds(cid * n + r0, rows_pt)])
        pltpu.sync_copy(accw.at[pl.ds(r0, rows_pt)],
                        outw_hbm.at[pl.ds(cid * n + r0, rows_pt)])

    return pl.kernel(
        body,
        out_type=(jax.ShapeDtypeStruct((2 * n, 128), jnp.float32),
                  jax.ShapeDtypeStruct((2 * n, 16), jnp.float32)),
        compiler_params=pltpu.CompilerParams(
            use_tc_tiling_on_sc=False, needs_layout_passes=False),
        mesh=plsc.VectorSubcoreMesh(core_axis_name="c", subcore_axis_name="s"),
        scratch_types=[
            pltpu.VMEM_SHARED((n, 128), jnp.float32),
            pltpu.VMEM_SHARED((n, 16), jnp.float32),
            pltpu.VMEM((16,), jnp.float32),
            pltpu.VMEM((B,), jnp.int32),
            pltpu.VMEM((B,), jnp.int32),
            pltpu.VMEM((B,), jnp.int32),
            pltpu.VMEM((B,), jnp.int32),
            pltpu.VMEM((B, 16), jnp.float32),
            pltpu.VMEM((B, 16), jnp.float32),
            pltpu.VMEM((B, 16), jnp.float32),
            pltpu.VMEM((B, 128), jnp.float32),
            pltpu.VMEM((B, 128), jnp.float32),
            pltpu.SemaphoreType.DMA,
            pltpu.SemaphoreType.DMA,
            pltpu.SemaphoreType.DMA,
        ],
    )


def _sc_edge1(n, np_, ka, kb):
    rows_pt = n // 16

    def body(src_hbm, dst_hbm, a1_hbm, d1_hbm, h1e_hbm, g_hbm, z80_hbm, out_hbm,
             accs, gv, a1v, d1v, six0, dix0, six1, dix1, wbuf,
             mbuf0, mbuf1, sh0, sh1):
        cid = lax.axis_index("c")
        sid = lax.axis_index("s")
        chunks = jnp.where(cid == 0, ka, kb)
        cbase = jnp.where(cid == 0, sid * ka, 16 * ka + sid * kb)
        r0 = sid * rows_pt

        pltpu.sync_copy(g_hbm, gv)
        pltpu.sync_copy(src_hbm.at[cbase], six0)
        pltpu.sync_copy(dst_hbm.at[cbase], dix0)
        pltpu.sync_copy(src_hbm.at[cbase + 1], six1)
        pltpu.sync_copy(dst_hbm.at[cbase + 1], dix1)
        pltpu.async_copy(h1e_hbm.at[six0], mbuf0, sh0)
        pltpu.async_copy(h1e_hbm.at[six1], mbuf1, sh1)
        pltpu.sync_copy(a1_hbm, a1v)
        pltpu.sync_copy(d1_hbm, d1v)

        pltpu.sync_copy(z80_hbm.at[pl.ds(r0, rows_pt)], accs.at[pl.ds(r0, rows_pt)])
        plsc.subcore_barrier()

        g = gv[...]

        def phase(c, sb, db, mb, sh):
            @plsc.parallel_loop(0, B // 16, unroll=2)
            def wcalc(q):
                srcv = sb[pl.ds(q * 16, 16)]
                dstv = db[pl.ds(q * 16, 16)]
                a = plsc.load_gather(a1v, [srcv])
                d = plsc.load_gather(d1v, [dstv])
                zz = a + d
                lz = jnp.where(zz > 0, zz, 0.2 * zz)
                m = jnp.maximum(g + d, 0.0)
                wbuf[pl.ds(q * 16, 16)] = jnp.exp(lz - m)

            pltpu.make_async_copy(h1e_hbm.at[sb], mb, sh).wait()

            @plsc.parallel_loop(0, B // 16)
            def mcalc(q):
                wvec = wbuf[pl.ds(q * 16, 16)]
                for i in range(16):
                    e = q * 16 + i
                    ws = wvec[i]
                    for j in range(5):
                        mb[e, pl.ds(j * 16, 16)] = ws * mb[e, pl.ds(j * 16, 16)]

            pltpu.sync_copy(mb, accs.at[db], add=True)

            @pl.when(c + 2 < chunks)
            def _():
                pltpu.sync_copy(src_hbm.at[cbase + c + 2], sb)
                pltpu.sync_copy(dst_hbm.at[cbase + c + 2], db)
                pltpu.async_copy(h1e_hbm.at[sb], mb, sh)

        def pair(p, _):
            c0 = 2 * p
            phase(c0, six0, dix0, mbuf0, sh0)
            phase(c0 + 1, six1, dix1, mbuf1, sh1)
            return 0
        lax.fori_loop(0, jnp.where(cid == 0, ka // 2, kb // 2), pair, 0)

        plsc.subcore_barrier()
        pltpu.sync_copy(accs.at[pl.ds(r0, rows_pt)],
                        out_hbm.at[pl.ds(cid * n + r0, rows_pt)])

    return pl.kernel(
        body,
        out_type=jax.ShapeDtypeStruct((2 * n, 80), jnp.float32),
        compiler_params=pltpu.CompilerParams(
            use_tc_tiling_on_sc=False, needs_layout_passes=False),
        mesh=plsc.VectorSubcoreMesh(core_axis_name="c", subcore_axis_name="s"),
        scratch_types=[
            pltpu.VMEM_SHARED((n, 80), jnp.float32),
            pltpu.VMEM((16,), jnp.float32),
            pltpu.VMEM((np_,), jnp.float32),
            pltpu.VMEM((np_,), jnp.float32),
            pltpu.VMEM((B,), jnp.int32),
            pltpu.VMEM((B,), jnp.int32),
            pltpu.VMEM((B,), jnp.int32),
            pltpu.VMEM((B,), jnp.int32),
            pltpu.VMEM((B,), jnp.float32),
            pltpu.VMEM((B, 80), jnp.float32),
            pltpu.VMEM((B, 80), jnp.float32),
            pltpu.SemaphoreType.DMA,
            pltpu.SemaphoreType.DMA,
        ],
    )


def kernel(x, edge_index, W0, a_src0, a_dst0, b0, W1, a_src1, a_dst1, b1):
    N, IN = x.shape
    HC = W0.shape[1]
    H = a_src0.shape[1]
    OC = W1.shape[1]
    f32 = jnp.float32

    np_ = ((N + 1023) // 1024) * 1024
    nblk = np_ // BLK
    nblk2 = N // BLK2

    ei = edge_index.astype(jnp.int32)
    loop = jnp.arange(N, dtype=jnp.int32)
    src = jnp.concatenate([ei[0], loop])
    dst = jnp.concatenate([ei[1], loop])
    etot = src.shape[0]
    step = 16 * B * 2 * KC
    ep = ((etot + step - 1) // step) * step
    tot = ep // (16 * B)

    def _split(frac):
        ka = min(tot - 2, max(2, int(round(tot * frac / 2)) * 2))
        return ka, tot - ka

    ka0, kb0 = _split(0.695)
    ka1, kb1 = _split(0.633)
    pad = ep - etot
    src = jnp.concatenate([src, jnp.full((pad,), np_ - 1, jnp.int32)]).reshape(-1, B)
    dst = jnp.concatenate([dst, jnp.zeros((pad,), jnp.int32)]).reshape(-1, B)

    xp = jnp.pad(x, ((0, np_ - N), (0, 0)))
    n_arr = jnp.array([N], jnp.int32)
    z128 = jnp.zeros((N, 128), f32)
    z16 = jnp.zeros((N, 16), f32)
    z80 = jnp.zeros((N, 80), f32)

    h0, U, V, g16 = pl.pallas_call(
        _tc_prep0,
        grid=(nblk,),
        in_specs=[
            pl.BlockSpec((BLK, IN), lambda i: (i, 0)),
            pl.BlockSpec((IN, HC), lambda i: (0, 0)),
            pl.BlockSpec((1, HC), lambda i: (0, 0)),
            pl.BlockSpec((1, HC), lambda i: (0, 0)),
            pl.BlockSpec(memory_space=pltpu.SMEM),
        ],
        out_specs=[
            pl.BlockSpec((BLK, HC), lambda i: (i, 0)),
            pl.BlockSpec((BLK, 16), lambda i: (i, 0)),
            pl.BlockSpec((BLK, 16), lambda i: (i, 0)),
            pl.BlockSpec((1, 16), lambda i: (0, 0)),
        ],
        out_shape=[
            jax.ShapeDtypeStruct((np_, HC), f32),
            jax.ShapeDtypeStruct((np_, 16), f32),
            jax.ShapeDtypeStruct((np_, 16), f32),
            jax.ShapeDtypeStruct((1, 16), f32),
        ],
        scratch_shapes=[pltpu.VMEM((8, 128), f32)],
    )(xp, W0, a_src0.reshape(1, HC), a_dst0.reshape(1, HC), n_arr)

    acch, accw = _sc_edge0(N, ka0, kb0)(
        src, dst, U, V, h0, g16.reshape(16), z128, z16)

    h1e, adt, g1 = pl.pallas_call(
        _tc_mid,
        grid=(nblk2,),
        in_specs=[
            pl.BlockSpec((BLK2, HC), lambda i: (i, 0)),
            pl.BlockSpec((BLK2, HC), lambda i, nb=nblk2: (i + nb, 0)),
            pl.BlockSpec((BLK2, 16), lambda i: (i, 0)),
            pl.BlockSpec((BLK2, 16), lambda i, nb=nblk2: (i + nb, 0)),
            pl.BlockSpec((1, HC), lambda i: (0, 0)),
            pl.BlockSpec((HC, OC), lambda i: (0, 0)),
            pl.BlockSpec((1, OC), lambda i: (0, 0)),
            pl.BlockSpec((1, OC), lambda i: (0, 0)),
        ],
        out_specs=[
            pl.BlockSpec((BLK2, 80), lambda i: (i, 0)),
            pl.BlockSpec((BLK2, 16), lambda i: (i, 0)),
            pl.BlockSpec((1, 16), lambda i: (0, 0)),
        ],
        out_shape=[
            jax.ShapeDtypeStruct((N, 80), f32),
            jax.ShapeDtypeStruct((N, 16), f32),
            jax.ShapeDtypeStruct((1, 16), f32),
        ],
        scratch_shapes=[pltpu.VMEM((8, 128), f32)],
    )(acch, acch, accw, accw, b0.reshape(1, HC), W1, a_src1.reshape(1, OC),
      a_dst1.reshape(1, OC))

    h1e_p = jnp.pad(h1e, ((0, np_ - N), (0, 0)))
    a1t = jnp.pad(adt[:, 0].reshape(N), (0, np_ - N), constant_values=NEG)
    d1t = jnp.pad(adt[:, 1].reshape(N), (0, np_ - N))
    acc1 = _sc_edge1(N, np_, ka1, kb1)(
        src, dst, a1t, d1t, h1e_p, g1.reshape(16), z80)

    out = pl.pallas_call(
        _tc_final,
        grid=(nblk2,),
        in_specs=[
            pl.BlockSpec((BLK2, 80), lambda i: (i, 0)),
            pl.BlockSpec((BLK2, 80), lambda i, nb=nblk2: (i + nb, 0)),
            pl.BlockSpec((1, OC), lambda i: (0, 0)),
        ],
        out_specs=pl.BlockSpec((BLK2, OC), lambda i: (i, 0)),
        out_shape=jax.ShapeDtypeStruct((N, OC), f32),
    )(acc1, acc1, b1.reshape(1, OC))

    return out


# speedup vs baseline: 1.0290x; 1.0010x over previous
import jax
import jax.numpy as jnp
from jax import lax
from jax.experimental import pallas as pl
from jax.experimental.pallas import tpu as pltpu
from jax.experimental.pallas import tpu_sc as plsc

NEG = -1.0e30
BIG = 1.0e30
B = 128
KC = 4
BLK = 512
BLK2 = 1000


def _tc_prep0(x_ref, w0_ref, as_ref, ad_ref, n_ref, h0_o, u_o, v_o, g_o, gsc):
    i = pl.program_id(0)
    blk = x_ref.shape[0]
    n_real = n_ref[0]
    h0 = jnp.dot(x_ref[...], w0_ref[...], preferred_element_type=jnp.float32)
    h0_o[...] = h0
    ps = h0 * as_ref[...]
    pd = h0 * ad_ref[...]
    H = u_o.shape[1] // 2
    C = h0.shape[1] // H
    asrc = jnp.concatenate(
        [jnp.sum(ps[:, h * C:(h + 1) * C], axis=1, keepdims=True) for h in range(H)], axis=1)
    adst = jnp.concatenate(
        [jnp.sum(pd[:, h * C:(h + 1) * C], axis=1, keepdims=True) for h in range(H)], axis=1)
    rowid = i * blk + lax.broadcasted_iota(jnp.int32, (blk, 1), 0)
    valid = rowid < n_real
    asrc = jnp.where(valid, asrc, NEG)
    adst = jnp.where(valid, adst, NEG)
    z8 = jnp.zeros((blk, H), jnp.float32)
    u_o[...] = jnp.concatenate([asrc, z8], axis=1)
    v_o[...] = jnp.concatenate([adst, z8], axis=1)

    @pl.when(i == 0)
    def _():
        gsc[...] = jnp.full((8, 128), NEG, jnp.float32)

    bm = jnp.max(asrc, axis=0, keepdims=True)
    gsc[0:1, 0:H] = jnp.maximum(gsc[0:1, 0:H], bm)
    g_o[...] = jnp.concatenate(
        [gsc[0:1, 0:H], jnp.full((1, 16 - H), BIG, jnp.float32)], axis=1)


def _tc_mid(a0_ref, a1_ref, wa0_ref, wa1_ref, b0_ref, w1_ref, as1_ref, ad1_ref,
            h1e_o, adt_o, g1_o, gsc):
    i = pl.program_id(0)
    blk = a0_ref.shape[0]
    s = a0_ref[...] + a1_ref[...]
    den = wa0_ref[...] + wa1_ref[...]
    HC = b0_ref.shape[1]
    H = wa0_ref.shape[1] // 2
    C = HC // H
    outs = []
    for h in range(H):
        outs.append(s[:, h * C:(h + 1) * C] / (den[:, h:h + 1] + 1e-16))
    z = jnp.concatenate(outs, axis=1) + b0_ref[...]
    hp = jnp.where(z > 0, z, jnp.exp(jnp.minimum(z, 0.0)) - 1.0)
    h1 = jnp.dot(hp, w1_ref[...], preferred_element_type=jnp.float32)
    as1 = jnp.sum(h1 * as1_ref[...], axis=1, keepdims=True)
    ad1 = jnp.sum(h1 * ad1_ref[...], axis=1, keepdims=True)
    h1e_o[...] = jnp.concatenate(
        [h1, jnp.ones((blk, 1), jnp.float32), jnp.zeros((blk, 15), jnp.float32)], axis=1)
    adt_o[...] = jnp.concatenate(
        [as1, ad1, jnp.zeros((blk, 14), jnp.float32)], axis=1)

    @pl.when(i == 0)
    def _():
        gsc[...] = jnp.full((8, 128), NEG, jnp.float32)

    gsc[0:1, 0:1] = jnp.maximum(gsc[0:1, 0:1], jnp.max(as1, axis=0, keepdims=True))
    g1_o[...] = jnp.broadcast_to(gsc[0:1, 0:1], (1, 16))


def _tc_final(a0_ref, a1_ref, b1_ref, out_o):
    s = a0_ref[...] + a1_ref[...]
    OC = b1_ref.shape[1]
    out_o[...] = s[:, 0:OC] / (s[:, OC:OC + 1] + 1e-16) + b1_ref[...]


def _sc_edge0(n, ka, kb):
    rows_pt = n // 16

    def body(src_hbm, dst_hbm, u_hbm, v_hbm, h0_hbm, g_hbm, z128_hbm, z16_hbm,
             outh_hbm, outw_hbm,
             acch, accw, gv, six0, dix0, six1, dix1, ubuf, vbuf, wbuf,
             hbuf0, hbuf1, suv, sh0, sh1):
        cid = lax.axis_index("c")
        sid = lax.axis_index("s")
        chunks = jnp.where(cid == 0, ka, kb)
        cbase = jnp.where(cid == 0, sid * ka, 16 * ka + sid * kb)
        r0 = sid * rows_pt

        pltpu.sync_copy(g_hbm, gv)
        pltpu.sync_copy(src_hbm.at[pl.ds(cbase * B, B)], six0)
        pltpu.sync_copy(dst_hbm.at[pl.ds(cbase * B, B)], dix0)
        pltpu.sync_copy(src_hbm.at[pl.ds((cbase + 1) * B, B)], six1)
        pltpu.sync_copy(dst_hbm.at[pl.ds((cbase + 1) * B, B)], dix1)
        pltpu.async_copy(u_hbm.at[six0], ubuf, suv)
        pltpu.async_copy(v_hbm.at[dix0], vbuf, suv)
        pltpu.async_copy(h0_hbm.at[six0], hbuf0, sh0)
        pltpu.async_copy(h0_hbm.at[six1], hbuf1, sh1)

        pltpu.sync_copy(z128_hbm.at[pl.ds(r0, rows_pt)], acch.at[pl.ds(r0, rows_pt)])
        pltpu.sync_copy(z16_hbm.at[pl.ds(r0, rows_pt)], accw.at[pl.ds(r0, rows_pt)])
        plsc.subcore_barrier()

        g = gv[...]

        def phase(c, sb, db, hb, sh, sb_o, db_o):
            pltpu.make_async_copy(u_hbm.at[sb], ubuf, suv).wait()
            pltpu.make_async_copy(v_hbm.at[db], vbuf, suv).wait()

            @plsc.parallel_loop(0, B, unroll=4)
            def wcalc(e):
                uz = ubuf[e]
                vz = vbuf[e]
                zz = uz + vz
                lz = jnp.where(zz > 0, zz, 0.2 * zz)
                m = jnp.maximum(g + vz, 0.0)
                wbuf[e] = jnp.exp(lz - m)

            @pl.when(c + 1 < chunks)
            def _():
                pltpu.async_copy(u_hbm.at[sb_o], ubuf, suv)
                pltpu.async_copy(v_hbm.at[db_o], vbuf, suv)

            pltpu.make_async_copy(h0_hbm.at[sb], hb, sh).wait()

            @plsc.parallel_loop(0, B, unroll=2)
            def mcalc(e):
                wvec = wbuf[e]
                for j in range(8):
                    hb[e, pl.ds(j * 16, 16)] = wvec[j] * hb[e, pl.ds(j * 16, 16)]

            pltpu.sync_copy(hb, acch.at[db], add=True)
            pltpu.sync_copy(wbuf, accw.at[db], add=True)

            @pl.when(c + 2 < chunks)
            def _():
                pltpu.sync_copy(src_hbm.at[pl.ds((cbase + c + 2) * B, B)], sb)
                pltpu.sync_copy(dst_hbm.at[pl.ds((cbase + c + 2) * B, B)], db)
                pltpu.async_copy(h0_hbm.at[sb], hb, sh)

        def pair(p, _):
            c0 = 2 * p
            phase(c0, six0, dix0, hbuf0, sh0, six1, dix1)
            phase(c0 + 1, six1, dix1, hbuf1, sh1, six0, dix0)
            return 0
        lax.fori_loop(0, jnp.where(cid == 0, ka // 2, kb // 2), pair, 0)

        plsc.subcore_barrier()
        pltpu.sync_copy(acch.at[pl.ds(r0, rows_pt)],
                        outh_hbm.at[pl.ds(cid * n + r0, rows_pt)])
        pltpu.sync_copy(accw.at[pl.ds(r0, rows_pt)],
                        outw_hbm.at[pl.ds(cid * n + r0, rows_pt)])

    return pl.kernel(
        body,
        out_type=(jax.ShapeDtypeStruct((2 * n, 128), jnp.float32),
                  jax.ShapeDtypeStruct((2 * n, 16), jnp.float32)),
        compiler_params=pltpu.CompilerParams(
            use_tc_tiling_on_sc=False, needs_layout_passes=False),
        mesh=plsc.VectorSubcoreMesh(core_axis_name="c", subcore_axis_name="s"),
        scratch_types=[
            pltpu.VMEM_SHARED((n, 128), jnp.float32),
            pltpu.VMEM_SHARED((n, 16), jnp.float32),
            pltpu.VMEM((16,), jnp.float32),
            pltpu.VMEM((B,), jnp.int32),
            pltpu.VMEM((B,), jnp.int32),
            pltpu.VMEM((B,), jnp.int32),
            pltpu.VMEM((B,), jnp.int32),
            pltpu.VMEM((B, 16), jnp.float32),
            pltpu.VMEM((B, 16), jnp.float32),
            pltpu.VMEM((B, 16), jnp.float32),
            pltpu.VMEM((B, 128), jnp.float32),
            pltpu.VMEM((B, 128), jnp.float32),
            pltpu.SemaphoreType.DMA,
            pltpu.SemaphoreType.DMA,
            pltpu.SemaphoreType.DMA,
        ],
    )


def _sc_edge1(n, np_, ka, kb):
    rows_pt = n // 16

    def body(src_hbm, dst_hbm, a1_hbm, d1_hbm, h1e_hbm, g_hbm, z80_hbm, out_hbm,
             accs, gv, a1v, d1v, six0, dix0, six1, dix1, wbuf,
             mbuf0, mbuf1, sh0, sh1):
        cid = lax.axis_index("c")
        sid = lax.axis_index("s")
        chunks = jnp.where(cid == 0, ka, kb)
        cbase = jnp.where(cid == 0, sid * ka, 16 * ka + sid * kb)
        r0 = sid * rows_pt

        pltpu.sync_copy(g_hbm, gv)
        pltpu.sync_copy(src_hbm.at[pl.ds(cbase * B, B)], six0)
        pltpu.sync_copy(dst_hbm.at[pl.ds(cbase * B, B)], dix0)
        pltpu.sync_copy(src_hbm.at[pl.ds((cbase + 1) * B, B)], six1)
        pltpu.sync_copy(dst_hbm.at[pl.ds((cbase + 1) * B, B)], dix1)
        pltpu.async_copy(h1e_hbm.at[six0], mbuf0, sh0)
        pltpu.async_copy(h1e_hbm.at[six1], mbuf1, sh1)
        pltpu.sync_copy(a1_hbm, a1v)
        pltpu.sync_copy(d1_hbm, d1v)

        pltpu.sync_copy(z80_hbm.at[pl.ds(r0, rows_pt)], accs.at[pl.ds(r0, rows_pt)])
        plsc.subcore_barrier()

        g = gv[...]

        def phase(c, sb, db, mb, sh):
            @plsc.parallel_loop(0, B // 16, unroll=2)
            def wcalc(q):
                srcv = sb[pl.ds(q * 16, 16)]
                dstv = db[pl.ds(q * 16, 16)]
                a = plsc.load_gather(a1v, [srcv])
                d = plsc.load_gather(d1v, [dstv])
                zz = a + d
                lz = jnp.where(zz > 0, zz, 0.2 * zz)
                m = jnp.maximum(g + d, 0.0)
                wbuf[pl.ds(q * 16, 16)] = jnp.exp(lz - m)

            pltpu.make_async_copy(h1e_hbm.at[sb], mb, sh).wait()

            @plsc.parallel_loop(0, B // 16)
            def mcalc(q):
                wvec = wbuf[pl.ds(q * 16, 16)]
                for i in range(16):
                    e = q * 16 + i
                    ws = wvec[i]
                    for j in range(5):
                        mb[e, pl.ds(j * 16, 16)] = ws * mb[e, pl.ds(j * 16, 16)]

            pltpu.sync_copy(mb, accs.at[db], add=True)

            @pl.when(c + 2 < chunks)
            def _():
                pltpu.sync_copy(src_hbm.at[pl.ds((cbase + c + 2) * B, B)], sb)
                pltpu.sync_copy(dst_hbm.at[pl.ds((cbase + c + 2) * B, B)], db)
                pltpu.async_copy(h1e_hbm.at[sb], mb, sh)

        def pair(p, _):
            c0 = 2 * p
            phase(c0, six0, dix0, mbuf0, sh0)
            phase(c0 + 1, six1, dix1, mbuf1, sh1)
            return 0
        lax.fori_loop(0, jnp.where(cid == 0, ka // 2, kb // 2), pair, 0)

        plsc.subcore_barrier()
        pltpu.sync_copy(accs.at[pl.ds(r0, rows_pt)],
                        out_hbm.at[pl.ds(cid * n + r0, rows_pt)])

    return pl.kernel(
        body,
        out_type=jax.ShapeDtypeStruct((2 * n, 80), jnp.float32),
        compiler_params=pltpu.CompilerParams(
            use_tc_tiling_on_sc=False, needs_layout_passes=False),
        mesh=plsc.VectorSubcoreMesh(core_axis_name="c", subcore_axis_name="s"),
        scratch_types=[
            pltpu.VMEM_SHARED((n, 80), jnp.float32),
            pltpu.VMEM((16,), jnp.float32),
            pltpu.VMEM((np_,), jnp.float32),
            pltpu.VMEM((np_,), jnp.float32),
            pltpu.VMEM((B,), jnp.int32),
            pltpu.VMEM((B,), jnp.int32),
            pltpu.VMEM((B,), jnp.int32),
            pltpu.VMEM((B,), jnp.int32),
            pltpu.VMEM((B,), jnp.float32),
            pltpu.VMEM((B, 80), jnp.float32),
            pltpu.VMEM((B, 80), jnp.float32),
            pltpu.SemaphoreType.DMA,
            pltpu.SemaphoreType.DMA,
        ],
    )


def kernel(x, edge_index, W0, a_src0, a_dst0, b0, W1, a_src1, a_dst1, b1):
    N, IN = x.shape
    HC = W0.shape[1]
    H = a_src0.shape[1]
    OC = W1.shape[1]
    f32 = jnp.float32

    np_ = ((N + 1023) // 1024) * 1024
    nblk = np_ // BLK
    nblk2 = N // BLK2

    ei = edge_index.astype(jnp.int32)
    loop = jnp.arange(N, dtype=jnp.int32)
    src = jnp.concatenate([ei[0], loop])
    dst = jnp.concatenate([ei[1], loop])
    etot = src.shape[0]
    step = 16 * B * 2 * KC
    ep = ((etot + step - 1) // step) * step
    tot = ep // (16 * B)

    def _split(frac):
        ka = min(tot - 2, max(2, int(round(tot * frac / 2)) * 2))
        return ka, tot - ka

    ka0, kb0 = _split(0.695)
    ka1, kb1 = _split(0.633)
    pad = ep - etot
    src = jnp.concatenate([src, jnp.full((pad,), np_ - 1, jnp.int32)])
    dst = jnp.concatenate([dst, jnp.zeros((pad,), jnp.int32)])

    xp = jnp.pad(x, ((0, np_ - N), (0, 0)))
    n_arr = jnp.array([N], jnp.int32)
    z128 = jnp.zeros((N, 128), f32)
    z16 = jnp.zeros((N, 16), f32)
    z80 = jnp.zeros((N, 80), f32)

    h0, U, V, g16 = pl.pallas_call(
        _tc_prep0,
        grid=(nblk,),
        in_specs=[
            pl.BlockSpec((BLK, IN), lambda i: (i, 0)),
            pl.BlockSpec((IN, HC), lambda i: (0, 0)),
            pl.BlockSpec((1, HC), lambda i: (0, 0)),
            pl.BlockSpec((1, HC), lambda i: (0, 0)),
            pl.BlockSpec(memory_space=pltpu.SMEM),
        ],
        out_specs=[
            pl.BlockSpec((BLK, HC), lambda i: (i, 0)),
            pl.BlockSpec((BLK, 16), lambda i: (i, 0)),
            pl.BlockSpec((BLK, 16), lambda i: (i, 0)),
            pl.BlockSpec((1, 16), lambda i: (0, 0)),
        ],
        out_shape=[
            jax.ShapeDtypeStruct((np_, HC), f32),
            jax.ShapeDtypeStruct((np_, 16), f32),
            jax.ShapeDtypeStruct((np_, 16), f32),
            jax.ShapeDtypeStruct((1, 16), f32),
        ],
        scratch_shapes=[pltpu.VMEM((8, 128), f32)],
    )(xp, W0, a_src0.reshape(1, HC), a_dst0.reshape(1, HC), n_arr)

    acch, accw = _sc_edge0(N, ka0, kb0)(
        src, dst, U, V, h0, g16.reshape(16), z128, z16)

    h1e, adt, g1 = pl.pallas_call(
        _tc_mid,
        grid=(nblk2,),
        in_specs=[
            pl.BlockSpec((BLK2, HC), lambda i: (i, 0)),
            pl.BlockSpec((BLK2, HC), lambda i, nb=nblk2: (i + nb, 0)),
            pl.BlockSpec((BLK2, 16), lambda i: (i, 0)),
            pl.BlockSpec((BLK2, 16), lambda i, nb=nblk2: (i + nb, 0)),
            pl.BlockSpec((1, HC), lambda i: (0, 0)),
            pl.BlockSpec((HC, OC), lambda i: (0, 0)),
            pl.BlockSpec((1, OC), lambda i: (0, 0)),
            pl.BlockSpec((1, OC), lambda i: (0, 0)),
        ],
        out_specs=[
            pl.BlockSpec((BLK2, 80), lambda i: (i, 0)),
            pl.BlockSpec((BLK2, 16), lambda i: (i, 0)),
            pl.BlockSpec((1, 16), lambda i: (0, 0)),
        ],
        out_shape=[
            jax.ShapeDtypeStruct((N, 80), f32),
            jax.ShapeDtypeStruct((N, 16), f32),
            jax.ShapeDtypeStruct((1, 16), f32),
        ],
        scratch_shapes=[pltpu.VMEM((8, 128), f32)],
    )(acch, acch, accw, accw, b0.reshape(1, HC), W1, a_src1.reshape(1, OC),
      a_dst1.reshape(1, OC))

    h1e_p = jnp.pad(h1e, ((0, np_ - N), (0, 0)))
    a1t = jnp.pad(adt[:, 0].reshape(N), (0, np_ - N), constant_values=NEG)
    d1t = jnp.pad(adt[:, 1].reshape(N), (0, np_ - N))
    acc1 = _sc_edge1(N, np_, ka1, kb1)(
        src, dst, a1t, d1t, h1e_p, g1.reshape(16), z80)

    out = pl.pallas_call(
        _tc_final,
        grid=(nblk2,),
        in_specs=[
            pl.BlockSpec((BLK2, 80), lambda i: (i, 0)),
            pl.BlockSpec((BLK2, 80), lambda i, nb=nblk2: (i + nb, 0)),
            pl.BlockSpec((1, OC), lambda i: (0, 0)),
        ],
        out_specs=pl.BlockSpec((BLK2, OC), lambda i: (i, 0)),
        out_shape=jax.ShapeDtypeStruct((N, OC), f32),
    )(acc1, acc1, b1.reshape(1, OC))

    return out


# speedup vs baseline: 2.6909x; 2.6151x over previous
import jax
import jax.numpy as jnp
from jax import lax
from jax.experimental import pallas as pl
from jax.experimental.pallas import tpu as pltpu
from jax.experimental.pallas import tpu_sc as plsc

NEG = -1.0e30
BIG = 1.0e30
B = 128
KC = 4
BLK = 512
BLK2 = 1000


def _tc_prep0(x_ref, w0_ref, as_ref, ad_ref, n_ref, h0_o, u_o, v_o, g_o, gsc):
    i = pl.program_id(0)
    blk = x_ref.shape[0]
    n_real = n_ref[0]
    h0 = jnp.dot(x_ref[...], w0_ref[...], preferred_element_type=jnp.float32)
    h0_o[...] = h0
    ps = h0 * as_ref[...]
    pd = h0 * ad_ref[...]
    H = u_o.shape[1] // 2
    C = h0.shape[1] // H
    asrc = jnp.concatenate(
        [jnp.sum(ps[:, h * C:(h + 1) * C], axis=1, keepdims=True) for h in range(H)], axis=1)
    adst = jnp.concatenate(
        [jnp.sum(pd[:, h * C:(h + 1) * C], axis=1, keepdims=True) for h in range(H)], axis=1)
    rowid = i * blk + lax.broadcasted_iota(jnp.int32, (blk, 1), 0)
    valid = rowid < n_real
    asrc = jnp.where(valid, asrc, NEG)
    adst = jnp.where(valid, adst, NEG)
    z8 = jnp.zeros((blk, H), jnp.float32)
    u_o[...] = jnp.concatenate([asrc, z8], axis=1)
    v_o[...] = jnp.concatenate([adst, z8], axis=1)

    @pl.when(i == 0)
    def _():
        gsc[...] = jnp.full((8, 128), NEG, jnp.float32)

    bm = jnp.max(asrc, axis=0, keepdims=True)
    gsc[0:1, 0:H] = jnp.maximum(gsc[0:1, 0:H], bm)
    g_o[...] = jnp.concatenate(
        [gsc[0:1, 0:H], jnp.full((1, 16 - H), BIG, jnp.float32)], axis=1)


def _tc_mid(a0_ref, a1_ref, wa0_ref, wa1_ref, b0_ref, w1_ref, as1_ref, ad1_ref,
            h1e_o, adt_o, g1_o, gsc):
    i = pl.program_id(0)
    blk = a0_ref.shape[0]
    s = a0_ref[...] + a1_ref[...]
    den = wa0_ref[...] + wa1_ref[...]
    HC = b0_ref.shape[1]
    H = wa0_ref.shape[1] // 2
    C = HC // H
    outs = []
    for h in range(H):
        outs.append(s[:, h * C:(h + 1) * C] / (den[:, h:h + 1] + 1e-16))
    z = jnp.concatenate(outs, axis=1) + b0_ref[...]
    hp = jnp.where(z > 0, z, jnp.exp(jnp.minimum(z, 0.0)) - 1.0)
    h1 = jnp.dot(hp, w1_ref[...], preferred_element_type=jnp.float32)
    as1 = jnp.sum(h1 * as1_ref[...], axis=1, keepdims=True)
    ad1 = jnp.sum(h1 * ad1_ref[...], axis=1, keepdims=True)
    h1e_o[...] = jnp.concatenate(
        [h1, jnp.ones((blk, 1), jnp.float32), jnp.zeros((blk, 15), jnp.float32)], axis=1)
    adt_o[...] = jnp.concatenate(
        [as1, ad1, jnp.zeros((blk, 14), jnp.float32)], axis=1)

    @pl.when(i == 0)
    def _():
        gsc[...] = jnp.full((8, 128), NEG, jnp.float32)

    gsc[0:1, 0:1] = jnp.maximum(gsc[0:1, 0:1], jnp.max(as1, axis=0, keepdims=True))
    g1_o[...] = jnp.broadcast_to(gsc[0:1, 0:1], (1, 16))


def _tc_final(a0_ref, a1_ref, b1_ref, out_o):
    s = a0_ref[...] + a1_ref[...]
    OC = b1_ref.shape[1]
    out_o[...] = s[:, 0:OC] / (s[:, OC:OC + 1] + 1e-16) + b1_ref[...]


def _sc_edge0(n, ka, kb):
    rows_pt = n // 16

    def body(src_hbm, dst_hbm, u_hbm, v_hbm, h0_hbm, g_hbm, z128_hbm, z16_hbm,
             outh_hbm, outw_hbm,
             acch, accw, gv, six0, dix0, six1, dix1, ubuf, vbuf, wbuf,
             hbuf0, hbuf1, suv, sh0, sh1):
        cid = lax.axis_index("c")
        sid = lax.axis_index("s")
        chunks = jnp.where(cid == 0, ka, kb)
        cbase = jnp.where(cid == 0, sid * ka, 16 * ka + sid * kb)
        r0 = sid * rows_pt

        pltpu.sync_copy(g_hbm, gv)
        pltpu.sync_copy(src_hbm.at[pl.ds(cbase * B, B)], six0)
        pltpu.sync_copy(dst_hbm.at[pl.ds(cbase * B, B)], dix0)
        pltpu.sync_copy(src_hbm.at[pl.ds((cbase + 1) * B, B)], six1)
        pltpu.sync_copy(dst_hbm.at[pl.ds((cbase + 1) * B, B)], dix1)
        pltpu.async_copy(u_hbm.at[six0], ubuf, suv)
        pltpu.async_copy(v_hbm.at[dix0], vbuf, suv)
        pltpu.async_copy(h0_hbm.at[six0], hbuf0, sh0)
        pltpu.async_copy(h0_hbm.at[six1], hbuf1, sh1)

        pltpu.sync_copy(z128_hbm.at[pl.ds(r0, rows_pt)], acch.at[pl.ds(r0, rows_pt)])
        pltpu.sync_copy(z16_hbm.at[pl.ds(r0, rows_pt)], accw.at[pl.ds(r0, rows_pt)])
        plsc.subcore_barrier()

        g = gv[...]

        def phase(c, sb, db, hb, sh, sb_o, db_o):
            pltpu.make_async_copy(u_hbm.at[sb], ubuf, suv).wait()
            pltpu.make_async_copy(v_hbm.at[db], vbuf, suv).wait()

            @plsc.parallel_loop(0, B, unroll=4)
            def wcalc(e):
                uz = ubuf[e]
                vz = vbuf[e]
                zz = uz + vz
                lz = jnp.where(zz > 0, zz, 0.2 * zz)
                m = jnp.maximum(g + vz, 0.0)
                wbuf[e] = jnp.exp(lz - m)

            @pl.when(c + 1 < chunks)
            def _():
                pltpu.async_copy(u_hbm.at[sb_o], ubuf, suv)
                pltpu.async_copy(v_hbm.at[db_o], vbuf, suv)

            pltpu.make_async_copy(h0_hbm.at[sb], hb, sh).wait()

            @plsc.parallel_loop(0, B, unroll=2)
            def mcalc(e):
                wvec = wbuf[e]
                for j in range(8):
                    hb[e, pl.ds(j * 16, 16)] = wvec[j] * hb[e, pl.ds(j * 16, 16)]

            pltpu.sync_copy(hb, acch.at[db], add=True)
            pltpu.sync_copy(wbuf, accw.at[db], add=True)

            @pl.when(c + 2 < chunks)
            def _():
                pltpu.sync_copy(src_hbm.at[pl.ds((cbase + c + 2) * B, B)], sb)
                pltpu.sync_copy(dst_hbm.at[pl.ds((cbase + c + 2) * B, B)], db)
                pltpu.async_copy(h0_hbm.at[sb], hb, sh)

        def pair(p, _):
            c0 = 2 * p
            phase(c0, six0, dix0, hbuf0, sh0, six1, dix1)
            phase(c0 + 1, six1, dix1, hbuf1, sh1, six0, dix0)
            return 0
        lax.fori_loop(0, jnp.where(cid == 0, ka // 2, kb // 2), pair, 0)

        plsc.subcore_barrier()
        pltpu.sync_copy(acch.at[pl.ds(r0, rows_pt)],
                        outh_hbm.at[pl.ds(cid * n + r0, rows_pt)])
        pltpu.sync_copy(accw.at[pl.ds(r0, rows_pt)],
                        outw_hbm.at[pl.ds(cid * n + r0, rows_pt)])

    return pl.kernel(
        body,
        out_type=(jax.ShapeDtypeStruct((2 * n, 128), jnp.float32),
                  jax.ShapeDtypeStruct((2 * n, 16), jnp.float32)),
        compiler_params=pltpu.CompilerParams(
            use_tc_tiling_on_sc=False, needs_layout_passes=False),
        mesh=plsc.VectorSubcoreMesh(core_axis_name="c", subcore_axis_name="s"),
        scratch_types=[
            pltpu.VMEM_SHARED((n, 128), jnp.float32),
            pltpu.VMEM_SHARED((n, 16), jnp.float32),
            pltpu.VMEM((16,), jnp.float32),
            pltpu.VMEM((B,), jnp.int32),
            pltpu.VMEM((B,), jnp.int32),
            pltpu.VMEM((B,), jnp.int32),
            pltpu.VMEM((B,), jnp.int32),
            pltpu.VMEM((B, 16), jnp.float32),
            pltpu.VMEM((B, 16), jnp.float32),
            pltpu.VMEM((B, 16), jnp.float32),
            pltpu.VMEM((B, 128), jnp.float32),
            pltpu.VMEM((B, 128), jnp.float32),
            pltpu.SemaphoreType.DMA,
            pltpu.SemaphoreType.DMA,
            pltpu.SemaphoreType.DMA,
        ],
    )


def _sc_edge1(n, np_, ka, kb):
    rows_pt = n // 16

    def body(src_hbm, dst_hbm, a1_hbm, d1_hbm, h1e_hbm, g_hbm, z80_hbm, out_hbm,
             accs, gv, a1v, d1v, six0, dix0, six1, dix1, wbuf,
             mbuf0, mbuf1, sh0, sh1):
        cid = lax.axis_index("c")
        sid = lax.axis_index("s")
        chunks = jnp.where(cid == 0, ka, kb)
        cbase = jnp.where(cid == 0, sid * ka, 16 * ka + sid * kb)
        r0 = sid * rows_pt

        pltpu.sync_copy(g_hbm, gv)
        pltpu.sync_copy(src_hbm.at[pl.ds(cbase * B, B)], six0)
        pltpu.sync_copy(dst_hbm.at[pl.ds(cbase * B, B)], dix0)
        pltpu.sync_copy(src_hbm.at[pl.ds((cbase + 1) * B, B)], six1)
        pltpu.sync_copy(dst_hbm.at[pl.ds((cbase + 1) * B, B)], dix1)
        pltpu.async_copy(h1e_hbm.at[six0], mbuf0, sh0)
        pltpu.async_copy(h1e_hbm.at[six1], mbuf1, sh1)
        pltpu.sync_copy(a1_hbm, a1v)
        pltpu.sync_copy(d1_hbm, d1v)

        pltpu.sync_copy(z80_hbm.at[pl.ds(r0, rows_pt)], accs.at[pl.ds(r0, rows_pt)])
        plsc.subcore_barrier()

        g = gv[...]

        def phase(c, sb, db, mb, sh):
            @plsc.parallel_loop(0, B // 16, unroll=2)
            def wcalc(q):
                srcv = sb[pl.ds(q * 16, 16)]
                dstv = db[pl.ds(q * 16, 16)]
                a = plsc.load_gather(a1v, [srcv])
                d = plsc.load_gather(d1v, [dstv])
                zz = a + d
                lz = jnp.where(zz > 0, zz, 0.2 * zz)
                m = jnp.maximum(g + d, 0.0)
                wbuf[pl.ds(q * 16, 16)] = jnp.exp(lz - m)

            pltpu.make_async_copy(h1e_hbm.at[sb], mb, sh).wait()

            @plsc.parallel_loop(0, B // 16)
            def mcalc(q):
                wvec = wbuf[pl.ds(q * 16, 16)]
                for i in range(16):
                    e = q * 16 + i
                    ws = wvec[i]
                    for j in range(5):
                        mb[e, pl.ds(j * 16, 16)] = ws * mb[e, pl.ds(j * 16, 16)]

            pltpu.sync_copy(mb, accs.at[db], add=True)

            @pl.when(c + 2 < chunks)
            def _():
                pltpu.sync_copy(src_hbm.at[pl.ds((cbase + c + 2) * B, B)], sb)
                pltpu.sync_copy(dst_hbm.at[pl.ds((cbase + c + 2) * B, B)], db)
                pltpu.async_copy(h1e_hbm.at[sb], mb, sh)

        def pair(p, _):
            c0 = 2 * p
            phase(c0, six0, dix0, mbuf0, sh0)
            phase(c0 + 1, six1, dix1, mbuf1, sh1)
            return 0
        lax.fori_loop(0, jnp.where(cid == 0, ka // 2, kb // 2), pair, 0)

        plsc.subcore_barrier()
        pltpu.sync_copy(accs.at[pl.ds(r0, rows_pt)],
                        out_hbm.at[pl.ds(cid * n + r0, rows_pt)])

    return pl.kernel(
        body,
        out_type=jax.ShapeDtypeStruct((2 * n, 80), jnp.float32),
        compiler_params=pltpu.CompilerParams(
            use_tc_tiling_on_sc=False, needs_layout_passes=False),
        mesh=plsc.VectorSubcoreMesh(core_axis_name="c", subcore_axis_name="s"),
        scratch_types=[
            pltpu.VMEM_SHARED((n, 80), jnp.float32),
            pltpu.VMEM((16,), jnp.float32),
            pltpu.VMEM((np_,), jnp.float32),
            pltpu.VMEM((np_,), jnp.float32),
            pltpu.VMEM((B,), jnp.int32),
            pltpu.VMEM((B,), jnp.int32),
            pltpu.VMEM((B,), jnp.int32),
            pltpu.VMEM((B,), jnp.int32),
            pltpu.VMEM((B,), jnp.float32),
            pltpu.VMEM((B, 80), jnp.float32),
            pltpu.VMEM((B, 80), jnp.float32),
            pltpu.SemaphoreType.DMA,
            pltpu.SemaphoreType.DMA,
        ],
    )


def kernel(x, edge_index, W0, a_src0, a_dst0, b0, W1, a_src1, a_dst1, b1):
    N, IN = x.shape
    HC = W0.shape[1]
    H = a_src0.shape[1]
    OC = W1.shape[1]
    f32 = jnp.float32

    np_ = ((N + 1023) // 1024) * 1024
    nblk = np_ // BLK
    nblk2 = N // BLK2

    ei = edge_index.astype(jnp.int32)
    loop = jnp.arange(N, dtype=jnp.int32)
    src = jnp.concatenate([ei[0], loop])
    dst = jnp.concatenate([ei[1], loop])
    etot = src.shape[0]
    step = 16 * B * 2 * KC
    ep = ((etot + step - 1) // step) * step
    tot = ep // (16 * B)

    def _split(frac):
        ka = min(tot - 2, max(2, int(round(tot * frac / 2)) * 2))
        return ka, tot - ka

    ka0, kb0 = _split(0.5)
    ka1, kb1 = _split(0.5)
    pad = ep - etot
    pidx = jnp.arange(pad, dtype=jnp.int32)
    src = jnp.concatenate([src, N + (pidx % (np_ - N))])
    dst = jnp.concatenate([dst, pidx % N])

    xp = jnp.pad(x, ((0, np_ - N), (0, 0)))
    n_arr = jnp.array([N], jnp.int32)
    z128 = jnp.zeros((N, 128), f32)
    z16 = jnp.zeros((N, 16), f32)
    z80 = jnp.zeros((N, 80), f32)

    h0, U, V, g16 = pl.pallas_call(
        _tc_prep0,
        grid=(nblk,),
        in_specs=[
            pl.BlockSpec((BLK, IN), lambda i: (i, 0)),
            pl.BlockSpec((IN, HC), lambda i: (0, 0)),
            pl.BlockSpec((1, HC), lambda i: (0, 0)),
            pl.BlockSpec((1, HC), lambda i: (0, 0)),
            pl.BlockSpec(memory_space=pltpu.SMEM),
        ],
        out_specs=[
            pl.BlockSpec((BLK, HC), lambda i: (i, 0)),
            pl.BlockSpec((BLK, 16), lambda i: (i, 0)),
            pl.BlockSpec((BLK, 16), lambda i: (i, 0)),
            pl.BlockSpec((1, 16), lambda i: (0, 0)),
        ],
        out_shape=[
            jax.ShapeDtypeStruct((np_, HC), f32),
            jax.ShapeDtypeStruct((np_, 16), f32),
            jax.ShapeDtypeStruct((np_, 16), f32),
            jax.ShapeDtypeStruct((1, 16), f32),
        ],
        scratch_shapes=[pltpu.VMEM((8, 128), f32)],
    )(xp, W0, a_src0.reshape(1, HC), a_dst0.reshape(1, HC), n_arr)

    acch, accw = _sc_edge0(N, ka0, kb0)(
        src, dst, U, V, h0, g16.reshape(16), z128, z16)

    h1e, adt, g1 = pl.pallas_call(
        _tc_mid,
        grid=(nblk2,),
        in_specs=[
            pl.BlockSpec((BLK2, HC), lambda i: (i, 0)),
            pl.BlockSpec((BLK2, HC), lambda i, nb=nblk2: (i + nb, 0)),
            pl.BlockSpec((BLK2, 16), lambda i: (i, 0)),
            pl.BlockSpec((BLK2, 16), lambda i, nb=nblk2: (i + nb, 0)),
            pl.BlockSpec((1, HC), lambda i: (0, 0)),
            pl.BlockSpec((HC, OC), lambda i: (0, 0)),
            pl.BlockSpec((1, OC), lambda i: (0, 0)),
            pl.BlockSpec((1, OC), lambda i: (0, 0)),
        ],
        out_specs=[
            pl.BlockSpec((BLK2, 80), lambda i: (i, 0)),
            pl.BlockSpec((BLK2, 16), lambda i: (i, 0)),
            pl.BlockSpec((1, 16), lambda i: (0, 0)),
        ],
        out_shape=[
            jax.ShapeDtypeStruct((N, 80), f32),
            jax.ShapeDtypeStruct((N, 16), f32),
            jax.ShapeDtypeStruct((1, 16), f32),
        ],
        scratch_shapes=[pltpu.VMEM((8, 128), f32)],
    )(acch, acch, accw, accw, b0.reshape(1, HC), W1, a_src1.reshape(1, OC),
      a_dst1.reshape(1, OC))

    h1e_p = jnp.pad(h1e, ((0, np_ - N), (0, 0)))
    a1t = jnp.pad(adt[:, 0].reshape(N), (0, np_ - N), constant_values=NEG)
    d1t = jnp.pad(adt[:, 1].reshape(N), (0, np_ - N))
    acc1 = _sc_edge1(N, np_, ka1, kb1)(
        src, dst, a1t, d1t, h1e_p, g1.reshape(16), z80)

    out = pl.pallas_call(
        _tc_final,
        grid=(nblk2,),
        in_specs=[
            pl.BlockSpec((BLK2, 80), lambda i: (i, 0)),
            pl.BlockSpec((BLK2, 80), lambda i, nb=nblk2: (i + nb, 0)),
            pl.BlockSpec((1, OC), lambda i: (0, 0)),
        ],
        out_specs=pl.BlockSpec((BLK2, OC), lambda i: (i, 0)),
        out_shape=jax.ShapeDtypeStruct((N, OC), f32),
    )(acc1, acc1, b1.reshape(1, OC))

    return out


# speedup vs baseline: 3.3693x; 1.2521x over previous
import jax
import jax.numpy as jnp
from jax import lax
from jax.experimental import pallas as pl
from jax.experimental.pallas import tpu as pltpu
from jax.experimental.pallas import tpu_sc as plsc

NEG = -1.0e30
BIG = 1.0e30
B = 128
KC = 4
BLK = 512
BLK2 = 1000


def _tc_prep0(x_ref, w0_ref, as_ref, ad_ref, n_ref, h0_o, u_o, v_o, g_o, gsc):
    i = pl.program_id(0)
    blk = x_ref.shape[0]
    n_real = n_ref[0]
    h0 = jnp.dot(x_ref[...], w0_ref[...], preferred_element_type=jnp.float32)
    h0_o[...] = h0
    ps = h0 * as_ref[...]
    pd = h0 * ad_ref[...]
    H = u_o.shape[1] // 2
    C = h0.shape[1] // H
    asrc = jnp.concatenate(
        [jnp.sum(ps[:, h * C:(h + 1) * C], axis=1, keepdims=True) for h in range(H)], axis=1)
    adst = jnp.concatenate(
        [jnp.sum(pd[:, h * C:(h + 1) * C], axis=1, keepdims=True) for h in range(H)], axis=1)
    rowid = i * blk + lax.broadcasted_iota(jnp.int32, (blk, 1), 0)
    valid = rowid < n_real
    asrc = jnp.where(valid, asrc, NEG)
    adst = jnp.where(valid, adst, NEG)
    z8 = jnp.zeros((blk, H), jnp.float32)
    u_o[...] = jnp.concatenate([asrc, z8], axis=1)
    v_o[...] = jnp.concatenate([adst, z8], axis=1)

    @pl.when(i == 0)
    def _():
        gsc[...] = jnp.full((8, 128), NEG, jnp.float32)

    bm = jnp.max(asrc, axis=0, keepdims=True)
    gsc[0:1, 0:H] = jnp.maximum(gsc[0:1, 0:H], bm)
    g_o[...] = jnp.concatenate(
        [gsc[0:1, 0:H], jnp.full((1, 16 - H), BIG, jnp.float32)], axis=1)


def _tc_mid(a0_ref, a1_ref, wa0_ref, wa1_ref, b0_ref, w1_ref, as1_ref, ad1_ref,
            h1e_o, adt_o, g1_o, gsc):
    i = pl.program_id(0)
    blk = a0_ref.shape[0]
    s = a0_ref[...] + a1_ref[...]
    den = wa0_ref[...] + wa1_ref[...]
    HC = b0_ref.shape[1]
    H = wa0_ref.shape[1] // 2
    C = HC // H
    outs = []
    for h in range(H):
        outs.append(s[:, h * C:(h + 1) * C] / (den[:, h:h + 1] + 1e-16))
    z = jnp.concatenate(outs, axis=1) + b0_ref[...]
    hp = jnp.where(z > 0, z, jnp.exp(jnp.minimum(z, 0.0)) - 1.0)
    h1 = jnp.dot(hp, w1_ref[...], preferred_element_type=jnp.float32)
    as1 = jnp.sum(h1 * as1_ref[...], axis=1, keepdims=True)
    ad1 = jnp.sum(h1 * ad1_ref[...], axis=1, keepdims=True)
    h1e_o[...] = jnp.concatenate(
        [h1, jnp.ones((blk, 1), jnp.float32), jnp.zeros((blk, 15), jnp.float32)], axis=1)
    adt_o[...] = jnp.concatenate(
        [as1, ad1, jnp.zeros((blk, 14), jnp.float32)], axis=1)

    @pl.when(i == 0)
    def _():
        gsc[...] = jnp.full((8, 128), NEG, jnp.float32)

    gsc[0:1, 0:1] = jnp.maximum(gsc[0:1, 0:1], jnp.max(as1, axis=0, keepdims=True))
    g1_o[...] = jnp.broadcast_to(gsc[0:1, 0:1], (1, 16))


def _tc_final(a0_ref, a1_ref, b1_ref, out_o):
    s = a0_ref[...] + a1_ref[...]
    OC = b1_ref.shape[1]
    out_o[...] = s[:, 0:OC] / (s[:, OC:OC + 1] + 1e-16) + b1_ref[...]


def _sc_edge0(n, ka, kb):
    rows_pt = n // 16

    ga, gb = ka // KC, kb // KC

    def body(src_hbm, dst_hbm, u_hbm, v_hbm, h0_hbm, g_hbm, z128_hbm, z16_hbm,
             outh_hbm, outw_hbm,
             acch, accw, gv, six0, dix0, six1, dix1, ubuf, vbuf, wbuf,
             hbuf0, hbuf1, suv, sh0, sh1, sgi):
        cid = lax.axis_index("c")
        sid = lax.axis_index("s")
        chunks = jnp.where(cid == 0, ka, kb)
        groups = jnp.where(cid == 0, ga, gb)
        cbase = jnp.where(cid == 0, sid * ka, 16 * ka + sid * kb)
        r0 = sid * rows_pt

        pltpu.sync_copy(g_hbm, gv)
        pltpu.sync_copy(src_hbm.at[pl.ds(cbase, KC)], six0)
        pltpu.sync_copy(dst_hbm.at[pl.ds(cbase, KC)], dix0)
        pltpu.async_copy(src_hbm.at[pl.ds(cbase + KC, KC)], six1, sgi)
        pltpu.async_copy(dst_hbm.at[pl.ds(cbase + KC, KC)], dix1, sgi)
        pltpu.async_copy(u_hbm.at[six0.at[0]], ubuf, suv)
        pltpu.async_copy(v_hbm.at[dix0.at[0]], vbuf, suv)
        pltpu.async_copy(h0_hbm.at[six0.at[0]], hbuf0, sh0)
        pltpu.async_copy(h0_hbm.at[six0.at[1]], hbuf1, sh1)

        pltpu.sync_copy(z128_hbm.at[pl.ds(r0, rows_pt)], acch.at[pl.ds(r0, rows_pt)])
        pltpu.sync_copy(z16_hbm.at[pl.ds(r0, rows_pt)], accw.at[pl.ds(r0, rows_pt)])
        plsc.subcore_barrier()

        g = gv[...]

        def phase(c, sb, db, hb, sh, sb1, db1, sb2):
            pltpu.make_async_copy(u_hbm.at[sb], ubuf, suv).wait()
            pltpu.make_async_copy(v_hbm.at[db], vbuf, suv).wait()

            @plsc.parallel_loop(0, B, unroll=4)
            def wcalc(e):
                uz = ubuf[e]
                vz = vbuf[e]
                zz = uz + vz
                lz = jnp.where(zz > 0, zz, 0.2 * zz)
                m = jnp.maximum(g + vz, 0.0)
                wbuf[e] = jnp.exp(lz - m)

            @pl.when(c + 1 < chunks)
            def _():
                pltpu.async_copy(u_hbm.at[sb1], ubuf, suv)
                pltpu.async_copy(v_hbm.at[db1], vbuf, suv)

            pltpu.make_async_copy(h0_hbm.at[sb], hb, sh).wait()

            @plsc.parallel_loop(0, B, unroll=2)
            def mcalc(e):
                wvec = wbuf[e]
                for j in range(8):
                    hb[e, pl.ds(j * 16, 16)] = wvec[j] * hb[e, pl.ds(j * 16, 16)]

            pltpu.sync_copy(hb, acch.at[db], add=True)
            pltpu.sync_copy(wbuf, accw.at[db], add=True)

            @pl.when(c + 2 < chunks)
            def _():
                pltpu.async_copy(h0_hbm.at[sb2], hb, sh)

        def group(gi, sixa, dixa, sixb, dixb):
            for j in range(KC):
                c = gi * KC + j
                if j == KC - 2:
                    @pl.when(gi + 1 < groups)
                    def _():
                        pltpu.make_async_copy(
                            src_hbm.at[pl.ds(cbase + (gi + 1) * KC, KC)], sixb, sgi).wait()
                        pltpu.make_async_copy(
                            dst_hbm.at[pl.ds(cbase + (gi + 1) * KC, KC)], dixb, sgi).wait()
                sb, db = sixa.at[j], dixa.at[j]
                sb1 = sixa.at[j + 1] if j + 1 < KC else sixb.at[0]
                db1 = dixa.at[j + 1] if j + 1 < KC else dixb.at[0]
                sb2 = sixa.at[j + 2] if j + 2 < KC else sixb.at[j + 2 - KC]
                hb, sh = (hbuf0, sh0) if j % 2 == 0 else (hbuf1, sh1)
                phase(c, sb, db, hb, sh, sb1, db1, sb2)

            @pl.when(gi + 2 < groups)
            def _():
                pltpu.async_copy(
                    src_hbm.at[pl.ds(cbase + (gi + 2) * KC, KC)], sixa, sgi)
                pltpu.async_copy(
                    dst_hbm.at[pl.ds(cbase + (gi + 2) * KC, KC)], dixa, sgi)

        def gpair(p, _):
            group(2 * p, six0, dix0, six1, dix1)
            group(2 * p + 1, six1, dix1, six0, dix0)
            return 0
        lax.fori_loop(0, jnp.where(cid == 0, ga // 2, gb // 2), gpair, 0)

        plsc.subcore_barrier()
        pltpu.sync_copy(acch.at[pl.ds(r0, rows_pt)],
                        outh_hbm.at[pl.ds(cid * n + r0, rows_pt)])
        pltpu.sync_copy(accw.at[pl.ds(r0, rows_pt)],
                        outw_hbm.at[pl.ds(cid * n + r0, rows_pt)])

    return pl.kernel(
        body,
        out_type=(jax.ShapeDtypeStruct((2 * n, 128), jnp.float32),
                  jax.ShapeDtypeStruct((2 * n, 16), jnp.float32)),
        compiler_params=pltpu.CompilerParams(
            use_tc_tiling_on_sc=False, needs_layout_passes=False),
        mesh=plsc.VectorSubcoreMesh(core_axis_name="c", subcore_axis_name="s"),
        scratch_types=[
            pltpu.VMEM_SHARED((n, 128), jnp.float32),
            pltpu.VMEM_SHARED((n, 16), jnp.float32),
            pltpu.VMEM((16,), jnp.float32),
            pltpu.VMEM((KC, B), jnp.int32),
            pltpu.VMEM((KC, B), jnp.int32),
            pltpu.VMEM((KC, B), jnp.int32),
            pltpu.VMEM((KC, B), jnp.int32),
            pltpu.VMEM((B, 16), jnp.float32),
            pltpu.VMEM((B, 16), jnp.float32),
            pltpu.VMEM((B, 16), jnp.float32),
            pltpu.VMEM((B, 128), jnp.float32),
            pltpu.VMEM((B, 128), jnp.float32),
            pltpu.SemaphoreType.DMA,
            pltpu.SemaphoreType.DMA,
            pltpu.SemaphoreType.DMA,
            pltpu.SemaphoreType.DMA,
        ],
    )


def _sc_edge1(n, np_, ka, kb):
    rows_pt = n // 16

    ga, gb = ka // KC, kb // KC

    def body(src_hbm, dst_hbm, a1_hbm, d1_hbm, h1e_hbm, g_hbm, z80_hbm, out_hbm,
             accs, gv, a1v, d1v, six0, dix0, six1, dix1, wbuf,
             mbuf0, mbuf1, sh0, sh1, sgi):
        cid = lax.axis_index("c")
        sid = lax.axis_index("s")
        chunks = jnp.where(cid == 0, ka, kb)
        groups = jnp.where(cid == 0, ga, gb)
        cbase = jnp.where(cid == 0, sid * ka, 16 * ka + sid * kb)
        r0 = sid * rows_pt

        pltpu.sync_copy(g_hbm, gv)
        pltpu.sync_copy(src_hbm.at[pl.ds(cbase, KC)], six0)
        pltpu.sync_copy(dst_hbm.at[pl.ds(cbase, KC)], dix0)
        pltpu.async_copy(src_hbm.at[pl.ds(cbase + KC, KC)], six1, sgi)
        pltpu.async_copy(dst_hbm.at[pl.ds(cbase + KC, KC)], dix1, sgi)
        pltpu.async_copy(h1e_hbm.at[six0.at[0]], mbuf0, sh0)
        pltpu.async_copy(h1e_hbm.at[six0.at[1]], mbuf1, sh1)
        pltpu.sync_copy(a1_hbm, a1v)
        pltpu.sync_copy(d1_hbm, d1v)

        pltpu.sync_copy(z80_hbm.at[pl.ds(r0, rows_pt)], accs.at[pl.ds(r0, rows_pt)])
        plsc.subcore_barrier()

        g = gv[...]

        def phase(c, sb, db, mb, sh, sb2):
            @plsc.parallel_loop(0, B // 16, unroll=2)
            def wcalc(q):
                srcv = sb[pl.ds(q * 16, 16)]
                dstv = db[pl.ds(q * 16, 16)]
                a = plsc.load_gather(a1v, [srcv])
                d = plsc.load_gather(d1v, [dstv])
                zz = a + d
                lz = jnp.where(zz > 0, zz, 0.2 * zz)
                m = jnp.maximum(g + d, 0.0)
                wbuf[pl.ds(q * 16, 16)] = jnp.exp(lz - m)

            pltpu.make_async_copy(h1e_hbm.at[sb], mb, sh).wait()

            @plsc.parallel_loop(0, B // 16)
            def mcalc(q):
                wvec = wbuf[pl.ds(q * 16, 16)]
                for i in range(16):
                    e = q * 16 + i
                    ws = wvec[i]
                    for j in range(5):
                        mb[e, pl.ds(j * 16, 16)] = ws * mb[e, pl.ds(j * 16, 16)]

            pltpu.sync_copy(mb, accs.at[db], add=True)

            @pl.when(c + 2 < chunks)
            def _():
                pltpu.async_copy(h1e_hbm.at[sb2], mb, sh)

        def group(gi, sixa, dixa, sixb, dixb):
            for j in range(KC):
                c = gi * KC + j
                if j == KC - 2:
                    @pl.when(gi + 1 < groups)
                    def _():
                        pltpu.make_async_copy(
                            src_hbm.at[pl.ds(cbase + (gi + 1) * KC, KC)], sixb, sgi).wait()
                        pltpu.make_async_copy(
                            dst_hbm.at[pl.ds(cbase + (gi + 1) * KC, KC)], dixb, sgi).wait()
                sb, db = sixa.at[j], dixa.at[j]
                sb2 = sixa.at[j + 2] if j + 2 < KC else sixb.at[j + 2 - KC]
                mb, sh = (mbuf0, sh0) if j % 2 == 0 else (mbuf1, sh1)
                phase(c, sb, db, mb, sh, sb2)

            @pl.when(gi + 2 < groups)
            def _():
                pltpu.async_copy(
                    src_hbm.at[pl.ds(cbase + (gi + 2) * KC, KC)], sixa, sgi)
                pltpu.async_copy(
                    dst_hbm.at[pl.ds(cbase + (gi + 2) * KC, KC)], dixa, sgi)

        def gpair(p, _):
            group(2 * p, six0, dix0, six1, dix1)
            group(2 * p + 1, six1, dix1, six0, dix0)
            return 0
        lax.fori_loop(0, jnp.where(cid == 0, ga // 2, gb // 2), gpair, 0)

        plsc.subcore_barrier()
        pltpu.sync_copy(accs.at[pl.ds(r0, rows_pt)],
                        out_hbm.at[pl.ds(cid * n + r0, rows_pt)])

    return pl.kernel(
        body,
        out_type=jax.ShapeDtypeStruct((2 * n, 80), jnp.float32),
        compiler_params=pltpu.CompilerParams(
            use_tc_tiling_on_sc=False, needs_layout_passes=False),
        mesh=plsc.VectorSubcoreMesh(core_axis_name="c", subcore_axis_name="s"),
        scratch_types=[
            pltpu.VMEM_SHARED((n, 80), jnp.float32),
            pltpu.VMEM((16,), jnp.float32),
            pltpu.VMEM((np_,), jnp.float32),
            pltpu.VMEM((np_,), jnp.float32),
            pltpu.VMEM((KC, B), jnp.int32),
            pltpu.VMEM((KC, B), jnp.int32),
            pltpu.VMEM((KC, B), jnp.int32),
            pltpu.VMEM((KC, B), jnp.int32),
            pltpu.VMEM((B,), jnp.float32),
            pltpu.VMEM((B, 80), jnp.float32),
            pltpu.VMEM((B, 80), jnp.float32),
            pltpu.SemaphoreType.DMA,
            pltpu.SemaphoreType.DMA,
            pltpu.SemaphoreType.DMA,
        ],
    )


def kernel(x, edge_index, W0, a_src0, a_dst0, b0, W1, a_src1, a_dst1, b1):
    N, IN = x.shape
    HC = W0.shape[1]
    H = a_src0.shape[1]
    OC = W1.shape[1]
    f32 = jnp.float32

    np_ = ((N + 1023) // 1024) * 1024
    nblk = np_ // BLK
    nblk2 = N // BLK2

    ei = edge_index.astype(jnp.int32)
    loop = jnp.arange(N, dtype=jnp.int32)
    src = jnp.concatenate([ei[0], loop])
    dst = jnp.concatenate([ei[1], loop])
    etot = src.shape[0]
    step = 16 * B * 2 * KC
    ep = ((etot + step - 1) // step) * step
    tot = ep // (16 * B)

    def _split(frac):
        m = 2 * KC
        ka = min(tot - m, max(m, int(round(tot * frac / m)) * m))
        return ka, tot - ka

    ka0, kb0 = _split(0.5)
    ka1, kb1 = _split(0.5)
    pad = ep - etot
    pidx = jnp.arange(pad, dtype=jnp.int32)
    src = jnp.concatenate([src, N + (pidx % (np_ - N))]).reshape(-1, B)
    dst = jnp.concatenate([dst, pidx % N]).reshape(-1, B)

    xp = jnp.pad(x, ((0, np_ - N), (0, 0)))
    n_arr = jnp.array([N], jnp.int32)
    z128 = jnp.zeros((N, 128), f32)
    z16 = jnp.zeros((N, 16), f32)
    z80 = jnp.zeros((N, 80), f32)

    h0, U, V, g16 = pl.pallas_call(
        _tc_prep0,
        grid=(nblk,),
        in_specs=[
            pl.BlockSpec((BLK, IN), lambda i: (i, 0)),
            pl.BlockSpec((IN, HC), lambda i: (0, 0)),
            pl.BlockSpec((1, HC), lambda i: (0, 0)),
            pl.BlockSpec((1, HC), lambda i: (0, 0)),
            pl.BlockSpec(memory_space=pltpu.SMEM),
        ],
        out_specs=[
            pl.BlockSpec((BLK, HC), lambda i: (i, 0)),
            pl.BlockSpec((BLK, 16), lambda i: (i, 0)),
            pl.BlockSpec((BLK, 16), lambda i: (i, 0)),
            pl.BlockSpec((1, 16), lambda i: (0, 0)),
        ],
        out_shape=[
            jax.ShapeDtypeStruct((np_, HC), f32),
            jax.ShapeDtypeStruct((np_, 16), f32),
            jax.ShapeDtypeStruct((np_, 16), f32),
            jax.ShapeDtypeStruct((1, 16), f32),
        ],
        scratch_shapes=[pltpu.VMEM((8, 128), f32)],
    )(xp, W0, a_src0.reshape(1, HC), a_dst0.reshape(1, HC), n_arr)

    acch, accw = _sc_edge0(N, ka0, kb0)(
        src, dst, U, V, h0, g16.reshape(16), z128, z16)

    h1e, adt, g1 = pl.pallas_call(
        _tc_mid,
        grid=(nblk2,),
        in_specs=[
            pl.BlockSpec((BLK2, HC), lambda i: (i, 0)),
            pl.BlockSpec((BLK2, HC), lambda i, nb=nblk2: (i + nb, 0)),
            pl.BlockSpec((BLK2, 16), lambda i: (i, 0)),
            pl.BlockSpec((BLK2, 16), lambda i, nb=nblk2: (i + nb, 0)),
            pl.BlockSpec((1, HC), lambda i: (0, 0)),
            pl.BlockSpec((HC, OC), lambda i: (0, 0)),
            pl.BlockSpec((1, OC), lambda i: (0, 0)),
            pl.BlockSpec((1, OC), lambda i: (0, 0)),
        ],
        out_specs=[
            pl.BlockSpec((BLK2, 80), lambda i: (i, 0)),
            pl.BlockSpec((BLK2, 16), lambda i: (i, 0)),
            pl.BlockSpec((1, 16), lambda i: (0, 0)),
        ],
        out_shape=[
            jax.ShapeDtypeStruct((N, 80), f32),
            jax.ShapeDtypeStruct((N, 16), f32),
            jax.ShapeDtypeStruct((1, 16), f32),
        ],
        scratch_shapes=[pltpu.VMEM((8, 128), f32)],
    )(acch, acch, accw, accw, b0.reshape(1, HC), W1, a_src1.reshape(1, OC),
      a_dst1.reshape(1, OC))

    h1e_p = jnp.pad(h1e, ((0, np_ - N), (0, 0)))
    a1t = jnp.pad(adt[:, 0].reshape(N), (0, np_ - N), constant_values=NEG)
    d1t = jnp.pad(adt[:, 1].reshape(N), (0, np_ - N))
    acc1 = _sc_edge1(N, np_, ka1, kb1)(
        src, dst, a1t, d1t, h1e_p, g1.reshape(16), z80)

    out = pl.pallas_call(
        _tc_final,
        grid=(nblk2,),
        in_specs=[
            pl.BlockSpec((BLK2, 80), lambda i: (i, 0)),
            pl.BlockSpec((BLK2, 80), lambda i, nb=nblk2: (i + nb, 0)),
            pl.BlockSpec((1, OC), lambda i: (0, 0)),
        ],
        out_specs=pl.BlockSpec((BLK2, OC), lambda i: (i, 0)),
        out_shape=jax.ShapeDtypeStruct((N, OC), f32),
    )(acc1, acc1, b1.reshape(1, OC))

    return out
